# Initial kernel scaffold; baseline (speedup 1.0000x reference)
import jax
import jax.numpy as jnp
from jax.experimental import pallas as pl


def kernel(x, edge_index, batch, W1_rel, W1_root, b1, a1, W2_rel, W2_root, b2, a2, W3_rel, W3_root, b3, W0, b0, Wout, bout):
    raise NotImplementedError("write your pallas kernel here")


# speedup vs baseline: 12.8462x; 12.8462x over previous
import functools

import jax
import jax.numpy as jnp
from jax import lax
from jax.experimental import pallas as pl
from jax.experimental.pallas import tpu as pltpu
from jax.experimental.pallas import tpu_sc as plsc

N = 100000
E = 1600000
F_IN = 16
H = 32
HH = 16
G = 64

NC = 2
NS = 16

SUP = 1024
E_PAD = 32 * 49 * SUP
ROWS128 = E_PAD // 128
EPW = E_PAD // 32
EPC = E_PAD // 16
SINK = N

TBL = 102400
ZCH = 640
OCH = 625

ACC = 16 * 72
BPAD = EPC

_mesh = plsc.VectorSubcoreMesh(
    core_axis_name="c", subcore_axis_name="s", num_cores=NC, num_subcores=NS)


def _fill_zero(ref, nrows, width):
  z = jnp.zeros((width,), jnp.float32)
  def body(i, carry):
    ref[i] = z
    return carry
  lax.fori_loop(0, nrows, body, 0)


def _make_segsum(edge_split):
  n_sup = (EPW if edge_split else EPC) // SUP

  @functools.partial(
      pl.kernel,
      mesh=_mesh,
      compiler_params=pltpu.CompilerParams(use_tc_tiling_on_sc=False),
      out_type=[jax.ShapeDtypeStruct((TBL, HH), jnp.float32),
                jax.ShapeDtypeStruct((TBL, HH), jnp.float32)],
      scratch_types=[
          pltpu.VMEM((8, 128), jnp.int32),
          pltpu.VMEM((8, 128), jnp.int32),
          pltpu.VMEM((8, 128, HH), jnp.float32),
          pltpu.VMEM((ZCH, HH), jnp.float32),
          pltpu.SemaphoreType.DMA,
          pltpu.VMEM_SHARED((TBL, HH), jnp.float32),
      ],
  )
  def k(tA, tB, src2d, dst2d, outA, outB, svec, dvec, rows, zbuf,
        gsem, table):
    c = lax.axis_index("c")
    s = lax.axis_index("s")

    _fill_zero(zbuf, ZCH, HH)
    for kk in range(10):
      pltpu.sync_copy(zbuf, table.at[pl.ds(s * 6400 + kk * ZCH, ZCH)])
    plsc.subcore_barrier()

    if edge_split:
      base128 = (s * NC + c) * (EPW // 128)
    else:
      base128 = s * (EPC // 128)

    def body(i, carry):
      rb = base128 + i * 8
      pltpu.sync_copy(src2d.at[pl.ds(rb, 8)], svec)
      pltpu.sync_copy(dst2d.at[pl.ds(rb, 8)], dvec)
      for j in range(8):
        @pl.when(c == 0)
        def _g0():
          pltpu.async_copy(tA.at[svec.at[j]], rows.at[j], gsem)
        @pl.when(c == 1)
        def _g1():
          pltpu.async_copy(tB.at[svec.at[j]], rows.at[j], gsem)
      for j in range(8):
        pltpu.make_async_copy(tA.at[svec.at[j]], rows.at[j], gsem).wait()
      for j in range(8):
        pltpu.sync_copy(rows.at[j], table.at[dvec.at[j]], add=True)
      return carry

    lax.fori_loop(0, n_sup, body, 0)
    plsc.subcore_barrier()

    for kk in range(10):
      off = s * 6400 + kk * ZCH
      pltpu.sync_copy(table.at[pl.ds(off, ZCH)], zbuf)
      @pl.when(c == 0)
      def _w0():
        pltpu.sync_copy(zbuf, outA.at[pl.ds(off, ZCH)])
      @pl.when(c == 1)
      def _w1():
        pltpu.sync_copy(zbuf, outB.at[pl.ds(off, ZCH)])

  return k


_segsum_edge = _make_segsum(True)
_segsum_col = _make_segsum(False)


@functools.partial(
    pl.kernel,
    mesh=_mesh,
    compiler_params=pltpu.CompilerParams(use_tc_tiling_on_sc=False),
    out_type=jax.ShapeDtypeStruct((2 * ACC, HH), jnp.float32),
    scratch_types=[
        pltpu.VMEM((8, 128), jnp.int32),
        pltpu.VMEM((8, 128), jnp.int32),
        pltpu.VMEM((8, 128, HH), jnp.float32),
        pltpu.VMEM((8, 128), jnp.int32),
        pltpu.VMEM((BPAD // 16,), jnp.int32),
        pltpu.VMEM((72, HH), jnp.float32),
        pltpu.SemaphoreType.DMA,
        pltpu.SemaphoreType.DMA,
        pltpu.VMEM_SHARED((BPAD,), jnp.int32),
        pltpu.VMEM_SHARED((ACC, HH), jnp.float32),
    ],
)
def _pool_sc(tA, tB, src2d, dst2d, batch_hbm, out, svec, dvec, rows, bdrow,
             bbuf, zobuf, gsem, bsem, batch_sp, acc):
  c = lax.axis_index("c")
  s = lax.axis_index("s")
  s72 = s * 72

  _fill_zero(zobuf, 72, HH)
  pltpu.sync_copy(zobuf, acc.at[pl.ds(s72, 72)])
  bch = BPAD // 16
  pltpu.sync_copy(batch_hbm.at[pl.ds(s * bch, bch)], bbuf)
  pltpu.sync_copy(bbuf, batch_sp.at[pl.ds(s * bch, bch)])
  plsc.subcore_barrier()

  base128 = s * (EPC // 128)

  def body(i, carry):
    rb = base128 + i * 8
    pltpu.sync_copy(src2d.at[pl.ds(rb, 8)], svec)
    pltpu.sync_copy(dst2d.at[pl.ds(rb, 8)], dvec)
    bd_handles = []
    for j in range(8):
      bd_handles.append(
          pltpu.async_copy(batch_sp.at[dvec.at[j]], bdrow.at[j], bsem))
    for j in range(8):
      @pl.when(c == 0)
      def _g0():
        pltpu.async_copy(tA.at[svec.at[j]], rows.at[j], gsem)
      @pl.when(c == 1)
      def _g1():
        pltpu.async_copy(tB.at[svec.at[j]], rows.at[j], gsem)
    for h in bd_handles:
      h.wait()
    for j in range(8):
      for q in range(8):
        bdrow[j, pl.ds(q * 16, 16)] = bdrow[j, pl.ds(q * 16, 16)] + s72
    for j in range(8):
      pltpu.make_async_copy(tA.at[svec.at[j]], rows.at[j], gsem).wait()
    for j in range(8):
      pltpu.sync_copy(rows.at[j], acc.at[bdrow.at[j]], add=True)
    return carry

  lax.fori_loop(0, EPC // SUP, body, 0)
  plsc.subcore_barrier()

  pltpu.sync_copy(acc.at[pl.ds(s72, 72)], zobuf)
  pltpu.sync_copy(zobuf, out.at[pl.ds(c * ACC + s72, 72)])


BLK = 1000
GRID = N // BLK


def _prelu(h, a):
  return jnp.maximum(h, 0.0) + a * jnp.minimum(h, 0.0)


def _tc1_body(p0, p1, xr, wr, wt, b, a, oA, oB):
  agg = p0[...] + p1[...]
  h = (jnp.dot(agg, wr[...], preferred_element_type=jnp.float32)
       + jnp.dot(xr[...], wt[...], preferred_element_type=jnp.float32)
       + b[...])
  h = _prelu(h, a[0, 0])
  oA[...] = h[:, :HH]
  oB[...] = h[:, HH:]


_row_spec = pl.BlockSpec((BLK, HH), lambda i: (i, 0))
_w1_spec = pl.BlockSpec((F_IN, H), lambda i: (0, 0))
_w2_spec = pl.BlockSpec((H, H), lambda i: (0, 0))
_b_spec = pl.BlockSpec((1, H), lambda i: (0, 0))
_a_spec = pl.BlockSpec((1, 1), lambda i: (0, 0))

_tc_layer1 = pl.pallas_call(
    _tc1_body,
    grid=(GRID,),
    in_specs=[_row_spec, _row_spec, _row_spec, _w1_spec, _w1_spec, _b_spec,
              _a_spec],
    out_specs=[_row_spec, _row_spec],
    out_shape=[jax.ShapeDtypeStruct((N, HH), jnp.float32),
               jax.ShapeDtypeStruct((N, HH), jnp.float32)],
)


def _tc2_body(aA, aB, hA, hB, wr, wt, b, a, oA, oB):
  h = (jnp.dot(aA[...], wr[:HH, :], preferred_element_type=jnp.float32)
       + jnp.dot(aB[...], wr[HH:, :], preferred_element_type=jnp.float32)
       + jnp.dot(hA[...], wt[:HH, :], preferred_element_type=jnp.float32)
       + jnp.dot(hB[...], wt[HH:, :], preferred_element_type=jnp.float32)
       + b[...])
  h = _prelu(h, a[0, 0])
  oA[...] = h[:, :HH]
  oB[...] = h[:, HH:]


_tc_layer2 = pl.pallas_call(
    _tc2_body,
    grid=(GRID,),
    in_specs=[_row_spec, _row_spec, _row_spec, _row_spec, _w2_spec, _w2_spec,
              _b_spec, _a_spec],
    out_specs=[_row_spec, _row_spec],
    out_shape=[jax.ShapeDtypeStruct((N, HH), jnp.float32),
               jax.ShapeDtypeStruct((N, HH), jnp.float32)],
)


def _tc_final_body(hA, hB, bt, u, w3r, w3t, b3, w0, b0, wout, bout, out,
                   s2_acc, cnt_acc):
  i = pl.program_id(0)

  @pl.when(i == 0)
  def _init():
    s2_acc[...] = jnp.zeros((G, H), jnp.float32)
    cnt_acc[...] = jnp.zeros((G, 1), jnp.float32)

  bb = bt[0, 0, :]
  iota = lax.broadcasted_iota(jnp.int32, (G, BLK), 0)
  oh = (iota == bb[None, :]).astype(jnp.float32)
  h2 = jnp.concatenate([hA[...], hB[...]], axis=1)
  s2_acc[...] += jnp.dot(oh, h2, preferred_element_type=jnp.float32)
  cnt_acc[...] += jnp.sum(oh, axis=1, keepdims=True)

  @pl.when(i == GRID - 1)
  def _fin():
    uu = u[...]
    u0 = jnp.zeros((G, HH), jnp.float32)
    u1 = jnp.zeros((G, HH), jnp.float32)
    for t in range(NS):
      u0 = u0 + uu[t * 72:t * 72 + G, :]
      u1 = u1 + uu[ACC + t * 72:ACC + t * 72 + G, :]
    U = jnp.concatenate([u0, u1], axis=1)
    cnt = cnt_acc[...]
    sums3 = (jnp.dot(U, w3r[...], preferred_element_type=jnp.float32)
             + jnp.dot(s2_acc[...], w3t[...],
                       preferred_element_type=jnp.float32)
             + cnt * b3[...])
    pooled = sums3 / jnp.maximum(cnt, 1.0)
    o = jnp.dot(pooled, w0[...], preferred_element_type=jnp.float32) + b0[...]
    o = jnp.dot(o, wout[...], preferred_element_type=jnp.float32) + bout[...]
    out[...] = o


_tc_final = pl.pallas_call(
    _tc_final_body,
    grid=(GRID,),
    in_specs=[
        _row_spec, _row_spec,
        pl.BlockSpec((1, 1, BLK), lambda i: (i, 0, 0)),
        pl.BlockSpec((2 * ACC, HH), lambda i: (0, 0)),
        _w2_spec, _w2_spec, _b_spec,
        _w2_spec, _b_spec,
        pl.BlockSpec((H, 1), lambda i: (0, 0)),
        _a_spec,
    ],
    out_specs=pl.BlockSpec((G, 1), lambda i: (0, 0)),
    out_shape=jax.ShapeDtypeStruct((G, 1), jnp.float32),
    scratch_shapes=[pltpu.VMEM((G, H), jnp.float32),
                    pltpu.VMEM((G, 1), jnp.float32)],
)


def kernel(x, edge_index, batch, W1_rel, W1_root, b1, a1, W2_rel, W2_root, b2,
           a2, W3_rel, W3_root, b3, W0, b0, Wout, bout):
  src = edge_index[0]
  dst = edge_index[1]
  pad = E_PAD - E
  srcp = jnp.concatenate([src, jnp.zeros((pad,), jnp.int32)]).reshape(
      ROWS128, 128)
  dstp = jnp.concatenate([dst, jnp.full((pad,), SINK, jnp.int32)]).reshape(
      ROWS128, 128)
  batchp = jnp.concatenate([batch, jnp.full((BPAD - N,), G, jnp.int32)])

  b1r = b1.reshape(1, H)
  b2r = b2.reshape(1, H)
  b3r = b3.reshape(1, H)
  b0r = b0.reshape(1, H)
  boutr = bout.reshape(1, 1)
  a1r = a1.reshape(1, 1)
  a2r = a2.reshape(1, 1)

  p0, p1 = _segsum_edge(x, x, srcp, dstp)
  h1A, h1B = _tc_layer1(p0, p1, x, W1_rel, W1_root, b1r, a1r)
  aggA, aggB = _segsum_col(h1A, h1B, srcp, dstp)
  h2A, h2B = _tc_layer2(aggA, aggB, h1A, h1B, W2_rel, W2_root, b2r, a2r)
  uacc = _pool_sc(h2A, h2B, srcp, dstp, batchp)
  out = _tc_final(h2A, h2B, batch.reshape(GRID, 1, BLK), uacc,
                  W3_rel, W3_root, b3r, W0, b0r, Wout, boutr)
  return out


# speedup vs baseline: 13.1870x; 1.0265x over previous
import functools

import jax
import jax.numpy as jnp
from jax import lax
from jax.experimental import pallas as pl
from jax.experimental.pallas import tpu as pltpu
from jax.experimental.pallas import tpu_sc as plsc

N = 100000
E = 1600000
F_IN = 16
H = 32
HH = 16
G = 64

NC = 2
NS = 16

SUP = 1024
E_PAD = 32 * 49 * SUP
ROWS128 = E_PAD // 128
EPW = E_PAD // 32
EPC = E_PAD // 16
SINK = N

TBL = 102400
ZCH = 320
OCH = 625

ACC = 16 * 72
BPAD = EPC

_mesh = plsc.VectorSubcoreMesh(
    core_axis_name="c", subcore_axis_name="s", num_cores=NC, num_subcores=NS)


def _fill_zero(ref, nrows, width):
  z = jnp.zeros((width,), jnp.float32)
  def body(i, carry):
    ref[i] = z
    return carry
  lax.fori_loop(0, nrows, body, 0)


def _make_segsum(edge_split):
  n_sup = (EPW if edge_split else EPC) // 512

  @functools.partial(
      pl.kernel,
      mesh=_mesh,
      compiler_params=pltpu.CompilerParams(use_tc_tiling_on_sc=False),
      out_type=[jax.ShapeDtypeStruct((TBL, HH), jnp.float32),
                jax.ShapeDtypeStruct((TBL, HH), jnp.float32)],
      scratch_types=[
          pltpu.VMEM((2, 4, 128), jnp.int32),
          pltpu.VMEM((2, 4, 128), jnp.int32),
          pltpu.VMEM((2, 4, 128, HH), jnp.float32),
          pltpu.VMEM((ZCH, HH), jnp.float32),
          pltpu.SemaphoreType.DMA,
          pltpu.SemaphoreType.DMA,
          pltpu.VMEM_SHARED((TBL, HH), jnp.float32),
      ],
  )
  def k(tA, tB, src2d, dst2d, outA, outB, svec, dvec, rows, zbuf,
        gsem, ssem, table):
    c = lax.axis_index("c")
    s = lax.axis_index("s")

    _fill_zero(zbuf, ZCH, HH)
    for kk in range(20):
      pltpu.sync_copy(zbuf, table.at[pl.ds(s * 6400 + kk * ZCH, ZCH)])
    plsc.subcore_barrier()

    if edge_split:
      base128 = (s * NC + c) * (EPW // 128)
    else:
      base128 = s * (EPC // 128)

    def stage_and_fire(p, i):
      rb = base128 + i * 4
      pltpu.sync_copy(src2d.at[pl.ds(rb, 4)], svec.at[p])
      pltpu.sync_copy(dst2d.at[pl.ds(rb, 4)], dvec.at[p])
      for j in range(4):
        @pl.when(c == 0)
        def _g0():
          pltpu.async_copy(tA.at[svec.at[p].at[j]], rows.at[p].at[j], gsem)
        @pl.when(c == 1)
        def _g1():
          pltpu.async_copy(tB.at[svec.at[p].at[j]], rows.at[p].at[j], gsem)

    def wait_gathers(p):
      for j in range(4):
        pltpu.make_async_copy(
            tA.at[svec.at[p].at[j]], rows.at[p].at[j], gsem).wait()

    def fire_scatters(p):
      for j in range(4):
        pltpu.async_copy(rows.at[p].at[j], table.at[dvec.at[p].at[j]], ssem,
                         add=True)

    def wait_scatters(p):
      for j in range(4):
        pltpu.make_async_copy(
            rows.at[p].at[j], table.at[dvec.at[p].at[j]], ssem).wait()

    def process(p, i):
      q = 1 - p
      wait_gathers(p)
      fire_scatters(p)
      @pl.when(i >= 1)
      def _dr():
        wait_scatters(q)
      @pl.when(i + 1 < n_sup)
      def _nx():
        stage_and_fire(q, i + 1)

    stage_and_fire(0, 0)

    def body(i, carry):
      @pl.when(i % 2 == 0)
      def _p0():
        process(0, i)
      @pl.when(i % 2 == 1)
      def _p1():
        process(1, i)
      return carry

    lax.fori_loop(0, n_sup, body, 0)
    wait_scatters((n_sup - 1) % 2)
    plsc.subcore_barrier()

    for kk in range(20):
      off = s * 6400 + kk * ZCH
      pltpu.sync_copy(table.at[pl.ds(off, ZCH)], zbuf)
      @pl.when(c == 0)
      def _w0():
        pltpu.sync_copy(zbuf, outA.at[pl.ds(off, ZCH)])
      @pl.when(c == 1)
      def _w1():
        pltpu.sync_copy(zbuf, outB.at[pl.ds(off, ZCH)])

  return k


_segsum_edge = _make_segsum(True)
_segsum_col = _make_segsum(False)


@functools.partial(
    pl.kernel,
    mesh=_mesh,
    compiler_params=pltpu.CompilerParams(use_tc_tiling_on_sc=False),
    out_type=jax.ShapeDtypeStruct((2 * ACC, HH), jnp.float32),
    scratch_types=[
        pltpu.VMEM((2, 8, 128), jnp.int32),
        pltpu.VMEM((2, 8, 128), jnp.int32),
        pltpu.VMEM((2, 8, 128, HH), jnp.float32),
        pltpu.VMEM((2, 8, 128), jnp.int32),
        pltpu.VMEM((BPAD // 16,), jnp.int32),
        pltpu.VMEM((72, HH), jnp.float32),
        pltpu.SemaphoreType.DMA,
        pltpu.SemaphoreType.DMA,
        pltpu.SemaphoreType.DMA,
        pltpu.VMEM_SHARED((BPAD,), jnp.int32),
        pltpu.VMEM_SHARED((ACC, HH), jnp.float32),
    ],
)
def _pool_sc(tA, tB, src2d, dst2d, batch_hbm, out, svec, dvec, rows, bdrow,
             bbuf, zobuf, gsem, bsem, ssem, batch_sp, acc):
  c = lax.axis_index("c")
  s = lax.axis_index("s")
  s72 = s * 72

  _fill_zero(zobuf, 72, HH)
  pltpu.sync_copy(zobuf, acc.at[pl.ds(s72, 72)])
  bch = BPAD // 16
  pltpu.sync_copy(batch_hbm.at[pl.ds(s * bch, bch)], bbuf)
  pltpu.sync_copy(bbuf, batch_sp.at[pl.ds(s * bch, bch)])
  plsc.subcore_barrier()

  base128 = s * (EPC // 128)
  n_sup = EPC // SUP

  def stage_and_fire(p, i):
    rb = base128 + i * 8
    pltpu.sync_copy(src2d.at[pl.ds(rb, 8)], svec.at[p])
    pltpu.sync_copy(dst2d.at[pl.ds(rb, 8)], dvec.at[p])
    for j in range(8):
      pltpu.async_copy(batch_sp.at[dvec.at[p].at[j]], bdrow.at[p].at[j], bsem)
    for j in range(8):
      @pl.when(c == 0)
      def _g0():
        pltpu.async_copy(tA.at[svec.at[p].at[j]], rows.at[p].at[j], gsem)
      @pl.when(c == 1)
      def _g1():
        pltpu.async_copy(tB.at[svec.at[p].at[j]], rows.at[p].at[j], gsem)

  def wait_scatters(p):
    for j in range(8):
      pltpu.make_async_copy(
          rows.at[p].at[j], acc.at[bdrow.at[p].at[j]], ssem).wait()

  def process(p, i):
    q = 1 - p
    for j in range(8):
      pltpu.make_async_copy(
          batch_sp.at[dvec.at[p].at[j]], bdrow.at[p].at[j], bsem).wait()
    for j in range(8):
      for qq in range(8):
        bdrow[p, j, pl.ds(qq * 16, 16)] = (
            bdrow[p, j, pl.ds(qq * 16, 16)] + s72)
    for j in range(8):
      pltpu.make_async_copy(
          tA.at[svec.at[p].at[j]], rows.at[p].at[j], gsem).wait()
    for j in range(8):
      pltpu.async_copy(rows.at[p].at[j], acc.at[bdrow.at[p].at[j]], ssem,
                       add=True)
    @pl.when(i >= 1)
    def _dr():
      wait_scatters(q)
    @pl.when(i + 1 < n_sup)
    def _nx():
      stage_and_fire(q, i + 1)

  stage_and_fire(0, 0)

  def body(i, carry):
    @pl.when(i % 2 == 0)
    def _p0():
      process(0, i)
    @pl.when(i % 2 == 1)
    def _p1():
      process(1, i)
    return carry

  lax.fori_loop(0, n_sup, body, 0)
  wait_scatters((n_sup - 1) % 2)
  plsc.subcore_barrier()

  pltpu.sync_copy(acc.at[pl.ds(s72, 72)], zobuf)
  pltpu.sync_copy(zobuf, out.at[pl.ds(c * ACC + s72, 72)])


BLK = 1000
GRID = N // BLK


def _prelu(h, a):
  return jnp.maximum(h, 0.0) + a * jnp.minimum(h, 0.0)


def _tc1_body(p0, p1, xr, wr, wt, b, a, oA, oB):
  agg = p0[...] + p1[...]
  h = (jnp.dot(agg, wr[...], preferred_element_type=jnp.float32)
       + jnp.dot(xr[...], wt[...], preferred_element_type=jnp.float32)
       + b[...])
  h = _prelu(h, a[0, 0])
  oA[...] = h[:, :HH]
  oB[...] = h[:, HH:]


_row_spec = pl.BlockSpec((BLK, HH), lambda i: (i, 0))
_w1_spec = pl.BlockSpec((F_IN, H), lambda i: (0, 0))
_w2_spec = pl.BlockSpec((H, H), lambda i: (0, 0))
_b_spec = pl.BlockSpec((1, H), lambda i: (0, 0))
_a_spec = pl.BlockSpec((1, 1), lambda i: (0, 0))

_tc_layer1 = pl.pallas_call(
    _tc1_body,
    grid=(GRID,),
    in_specs=[_row_spec, _row_spec, _row_spec, _w1_spec, _w1_spec, _b_spec,
              _a_spec],
    out_specs=[_row_spec, _row_spec],
    out_shape=[jax.ShapeDtypeStruct((N, HH), jnp.float32),
               jax.ShapeDtypeStruct((N, HH), jnp.float32)],
)


def _tc2_body(aA, aB, hA, hB, wr, wt, b, a, oA, oB):
  h = (jnp.dot(aA[...], wr[:HH, :], preferred_element_type=jnp.float32)
       + jnp.dot(aB[...], wr[HH:, :], preferred_element_type=jnp.float32)
       + jnp.dot(hA[...], wt[:HH, :], preferred_element_type=jnp.float32)
       + jnp.dot(hB[...], wt[HH:, :], preferred_element_type=jnp.float32)
       + b[...])
  h = _prelu(h, a[0, 0])
  oA[...] = h[:, :HH]
  oB[...] = h[:, HH:]


_tc_layer2 = pl.pallas_call(
    _tc2_body,
    grid=(GRID,),
    in_specs=[_row_spec, _row_spec, _row_spec, _row_spec, _w2_spec, _w2_spec,
              _b_spec, _a_spec],
    out_specs=[_row_spec, _row_spec],
    out_shape=[jax.ShapeDtypeStruct((N, HH), jnp.float32),
               jax.ShapeDtypeStruct((N, HH), jnp.float32)],
)


def _tc_final_body(hA, hB, bt, u, w3r, w3t, b3, w0, b0, wout, bout, out,
                   s2_acc, cnt_acc):
  i = pl.program_id(0)

  @pl.when(i == 0)
  def _init():
    s2_acc[...] = jnp.zeros((G, H), jnp.float32)
    cnt_acc[...] = jnp.zeros((G, 1), jnp.float32)

  bb = bt[0, 0, :]
  iota = lax.broadcasted_iota(jnp.int32, (G, BLK), 0)
  oh = (iota == bb[None, :]).astype(jnp.float32)
  h2 = jnp.concatenate([hA[...], hB[...]], axis=1)
  s2_acc[...] += jnp.dot(oh, h2, preferred_element_type=jnp.float32)
  cnt_acc[...] += jnp.sum(oh, axis=1, keepdims=True)

  @pl.when(i == GRID - 1)
  def _fin():
    uu = u[...]
    u0 = jnp.zeros((G, HH), jnp.float32)
    u1 = jnp.zeros((G, HH), jnp.float32)
    for t in range(NS):
      u0 = u0 + uu[t * 72:t * 72 + G, :]
      u1 = u1 + uu[ACC + t * 72:ACC + t * 72 + G, :]
    U = jnp.concatenate([u0, u1], axis=1)
    cnt = cnt_acc[...]
    sums3 = (jnp.dot(U, w3r[...], preferred_element_type=jnp.float32)
             + jnp.dot(s2_acc[...], w3t[...],
                       preferred_element_type=jnp.float32)
             + cnt * b3[...])
    pooled = sums3 / jnp.maximum(cnt, 1.0)
    o = jnp.dot(pooled, w0[...], preferred_element_type=jnp.float32) + b0[...]
    o = jnp.dot(o, wout[...], preferred_element_type=jnp.float32) + bout[...]
    out[...] = o


_tc_final = pl.pallas_call(
    _tc_final_body,
    grid=(GRID,),
    in_specs=[
        _row_spec, _row_spec,
        pl.BlockSpec((1, 1, BLK), lambda i: (i, 0, 0)),
        pl.BlockSpec((2 * ACC, HH), lambda i: (0, 0)),
        _w2_spec, _w2_spec, _b_spec,
        _w2_spec, _b_spec,
        pl.BlockSpec((H, 1), lambda i: (0, 0)),
        _a_spec,
    ],
    out_specs=pl.BlockSpec((G, 1), lambda i: (0, 0)),
    out_shape=jax.ShapeDtypeStruct((G, 1), jnp.float32),
    scratch_shapes=[pltpu.VMEM((G, H), jnp.float32),
                    pltpu.VMEM((G, 1), jnp.float32)],
)


def kernel(x, edge_index, batch, W1_rel, W1_root, b1, a1, W2_rel, W2_root, b2,
           a2, W3_rel, W3_root, b3, W0, b0, Wout, bout):
  src = edge_index[0]
  dst = edge_index[1]
  pad = E_PAD - E
  srcp = jnp.concatenate([src, jnp.zeros((pad,), jnp.int32)]).reshape(
      ROWS128, 128)
  dstp = jnp.concatenate([dst, jnp.full((pad,), SINK, jnp.int32)]).reshape(
      ROWS128, 128)
  batchp = jnp.concatenate([batch, jnp.full((BPAD - N,), G, jnp.int32)])

  b1r = b1.reshape(1, H)
  b2r = b2.reshape(1, H)
  b3r = b3.reshape(1, H)
  b0r = b0.reshape(1, H)
  boutr = bout.reshape(1, 1)
  a1r = a1.reshape(1, 1)
  a2r = a2.reshape(1, 1)

  p0, p1 = _segsum_edge(x, x, srcp, dstp)
  h1A, h1B = _tc_layer1(p0, p1, x, W1_rel, W1_root, b1r, a1r)
  aggA, aggB = _segsum_col(h1A, h1B, srcp, dstp)
  h2A, h2B = _tc_layer2(aggA, aggB, h1A, h1B, W2_rel, W2_root, b2r, a2r)
  uacc = _pool_sc(h2A, h2B, srcp, dstp, batchp)
  out = _tc_final(h2A, h2B, batch.reshape(GRID, 1, BLK), uacc,
                  W3_rel, W3_root, b3r, W0, b0r, Wout, boutr)
  return out


# speedup vs baseline: 18.0064x; 1.3655x over previous
import functools

import jax
import jax.numpy as jnp
from jax import lax
from jax.experimental import pallas as pl
from jax.experimental.pallas import tpu as pltpu
from jax.experimental.pallas import tpu_sc as plsc

N = 100000
E = 1600000
F_IN = 16
H = 32
HH = 16
G = 64

NC = 2
NS = 16

E_PAD = 32 * 49 * 1024
ROWS128 = E_PAD // 128
EPW = E_PAD // 32
EPC = E_PAD // 16
SINK = N

TBL = 102400
ZCH = 320

TPT = 72
REG = NS * TPT
ACC3 = 3 * REG
BPAD = EPC
NPT = BPAD // 16

_mesh = plsc.VectorSubcoreMesh(
    core_axis_name="c", subcore_axis_name="s", num_cores=NC, num_subcores=NS)


def _fill(ref, nrows, width, val):
  v = jnp.full((width,), val, jnp.float32)
  def body(i, carry):
    ref[i] = v
    return carry
  lax.fori_loop(0, nrows, body, 0)


def _make_segsum(edge_split):
  n_sup = (EPW if edge_split else EPC) // 512

  @functools.partial(
      pl.kernel,
      mesh=_mesh,
      compiler_params=pltpu.CompilerParams(use_tc_tiling_on_sc=False),
      out_type=[jax.ShapeDtypeStruct((TBL, HH), jnp.float32),
                jax.ShapeDtypeStruct((TBL, HH), jnp.float32)],
      scratch_types=[
          pltpu.VMEM((2, 4, 128), jnp.int32),
          pltpu.VMEM((2, 4, 128), jnp.int32),
          pltpu.VMEM((2, 4, 128, HH), jnp.float32),
          pltpu.VMEM((ZCH, HH), jnp.float32),
          pltpu.SemaphoreType.DMA,
          pltpu.SemaphoreType.DMA,
          pltpu.VMEM_SHARED((TBL, HH), jnp.float32),
      ],
  )
  def k(tA, tB, src2d, dst2d, outA, outB, svec, dvec, rows, zbuf,
        gsem, ssem, table):
    c = lax.axis_index("c")
    s = lax.axis_index("s")

    _fill(zbuf, ZCH, HH, 0.0)
    for kk in range(20):
      pltpu.sync_copy(zbuf, table.at[pl.ds(s * 6400 + kk * ZCH, ZCH)])
    plsc.subcore_barrier()

    if edge_split:
      base128 = (s * NC + c) * (EPW // 128)
    else:
      base128 = s * (EPC // 128)

    def stage_and_fire(p, i):
      rb = base128 + i * 4
      pltpu.sync_copy(src2d.at[pl.ds(rb, 4)], svec.at[p])
      pltpu.sync_copy(dst2d.at[pl.ds(rb, 4)], dvec.at[p])
      for j in range(4):
        @pl.when(c == 0)
        def _g0():
          pltpu.async_copy(tA.at[svec.at[p].at[j]], rows.at[p].at[j], gsem)
        @pl.when(c == 1)
        def _g1():
          pltpu.async_copy(tB.at[svec.at[p].at[j]], rows.at[p].at[j], gsem)

    def wait_gathers(p):
      for j in range(4):
        pltpu.make_async_copy(
            tA.at[svec.at[p].at[j]], rows.at[p].at[j], gsem).wait()

    def fire_scatters(p):
      for j in range(4):
        pltpu.async_copy(rows.at[p].at[j], table.at[dvec.at[p].at[j]], ssem,
                         add=True)

    def wait_scatters(p):
      for j in range(4):
        pltpu.make_async_copy(
            rows.at[p].at[j], table.at[dvec.at[p].at[j]], ssem).wait()

    def process(p, i):
      q = 1 - p
      wait_gathers(p)
      fire_scatters(p)
      @pl.when(i >= 1)
      def _dr():
        wait_scatters(q)
      @pl.when(i + 1 < n_sup)
      def _nx():
        stage_and_fire(q, i + 1)

    stage_and_fire(0, 0)

    def body(i, carry):
      @pl.when(i % 2 == 0)
      def _p0():
        process(0, i)
      @pl.when(i % 2 == 1)
      def _p1():
        process(1, i)
      return carry

    lax.fori_loop(0, n_sup, body, 0)
    wait_scatters((n_sup - 1) % 2)
    plsc.subcore_barrier()

    for kk in range(20):
      off = s * 6400 + kk * ZCH
      pltpu.sync_copy(table.at[pl.ds(off, ZCH)], zbuf)
      @pl.when(c == 0)
      def _w0():
        pltpu.sync_copy(zbuf, outA.at[pl.ds(off, ZCH)])
      @pl.when(c == 1)
      def _w1():
        pltpu.sync_copy(zbuf, outB.at[pl.ds(off, ZCH)])

  return k


_segsum_edge = _make_segsum(True)
_segsum_col = _make_segsum(False)


@functools.partial(
    pl.kernel,
    mesh=_mesh,
    compiler_params=pltpu.CompilerParams(use_tc_tiling_on_sc=False),
    out_type=jax.ShapeDtypeStruct((2 * ACC3, HH), jnp.float32),
    scratch_types=[
        pltpu.VMEM((2, 8, 128), jnp.int32),
        pltpu.VMEM((2, 8, 128), jnp.int32),
        pltpu.VMEM((2, 8, 128, HH), jnp.float32),
        pltpu.VMEM((2, 8, 128), jnp.int32),
        pltpu.VMEM((NPT,), jnp.int32),
        pltpu.VMEM((TPT, HH), jnp.float32),
        pltpu.VMEM((128, HH), jnp.float32),
        pltpu.VMEM((128, HH), jnp.float32),
        pltpu.VMEM((128,), jnp.int32),
        pltpu.SemaphoreType.DMA,
        pltpu.SemaphoreType.DMA,
        pltpu.SemaphoreType.DMA,
        pltpu.VMEM_SHARED((BPAD,), jnp.int32),
        pltpu.VMEM_SHARED((ACC3, HH), jnp.float32),
    ],
)
def _pool_sc(tA, tB, src2d, dst2d, batch_hbm, out, svec, dvec, rows, bdrow,
             bbuf, zobuf, nrow, ones, nidx, gsem, bsem, ssem, batch_sp, acc):
  c = lax.axis_index("c")
  s = lax.axis_index("s")
  s72 = s * TPT

  _fill(zobuf, TPT, HH, 0.0)
  for r in range(3):
    pltpu.sync_copy(zobuf, acc.at[pl.ds(r * REG + s72, TPT)])
  _fill(ones, 128, HH, 1.0)
  pltpu.sync_copy(batch_hbm.at[pl.ds(s * NPT, NPT)], bbuf)
  pltpu.sync_copy(bbuf, batch_sp.at[pl.ds(s * NPT, NPT)])
  plsc.subcore_barrier()

  base128 = s * (EPC // 128)
  n_sup = EPC // 1024

  def stage_and_fire(p, i):
    rb = base128 + i * 8
    pltpu.sync_copy(src2d.at[pl.ds(rb, 8)], svec.at[p])
    pltpu.sync_copy(dst2d.at[pl.ds(rb, 8)], dvec.at[p])
    for j in range(8):
      pltpu.async_copy(batch_sp.at[dvec.at[p].at[j]], bdrow.at[p].at[j], bsem)
    for j in range(8):
      @pl.when(c == 0)
      def _g0():
        pltpu.async_copy(tA.at[svec.at[p].at[j]], rows.at[p].at[j], gsem)
      @pl.when(c == 1)
      def _g1():
        pltpu.async_copy(tB.at[svec.at[p].at[j]], rows.at[p].at[j], gsem)

  def wait_scatters(p):
    for j in range(8):
      pltpu.make_async_copy(
          rows.at[p].at[j], acc.at[bdrow.at[p].at[j]], ssem).wait()

  def process(p, i):
    q = 1 - p
    for j in range(8):
      pltpu.make_async_copy(
          batch_sp.at[dvec.at[p].at[j]], bdrow.at[p].at[j], bsem).wait()
    for j in range(8):
      for qq in range(8):
        bdrow[p, j, pl.ds(qq * 16, 16)] = (
            bdrow[p, j, pl.ds(qq * 16, 16)] + s72)
    for j in range(8):
      pltpu.make_async_copy(
          tA.at[svec.at[p].at[j]], rows.at[p].at[j], gsem).wait()
    for j in range(8):
      pltpu.async_copy(rows.at[p].at[j], acc.at[bdrow.at[p].at[j]], ssem,
                       add=True)
    @pl.when(i >= 1)
    def _dr():
      wait_scatters(q)
    @pl.when(i + 1 < n_sup)
    def _nx():
      stage_and_fire(q, i + 1)

  stage_and_fire(0, 0)

  def body(i, carry):
    @pl.when(i % 2 == 0)
    def _p0():
      process(0, i)
    @pl.when(i % 2 == 1)
    def _p1():
      process(1, i)
    return carry

  lax.fori_loop(0, n_sup, body, 0)
  wait_scatters((n_sup - 1) % 2)

  def npass(k, carry):
    nb = s * NPT + k * 128
    @pl.when(c == 0)
    def _n0():
      pltpu.sync_copy(tA.at[pl.ds(nb, 128)], nrow)
    @pl.when(c == 1)
    def _n1():
      pltpu.sync_copy(tB.at[pl.ds(nb, 128)], nrow)
    for q in range(8):
      nidx[pl.ds(q * 16, 16)] = (
          bbuf[pl.ds(k * 128 + q * 16, 16)] + (REG + s72))
    pltpu.sync_copy(nrow, acc.at[nidx], add=True)
    for q in range(8):
      nidx[pl.ds(q * 16, 16)] = nidx[pl.ds(q * 16, 16)] + REG
    pltpu.sync_copy(ones, acc.at[nidx], add=True)
    return carry

  lax.fori_loop(0, NPT // 128, npass, 0)
  plsc.subcore_barrier()

  for r in range(3):
    pltpu.sync_copy(acc.at[pl.ds(r * REG + s72, TPT)], zobuf)
    pltpu.sync_copy(zobuf, out.at[pl.ds(c * ACC3 + r * REG + s72, TPT)])


PBLK = 1280
PGRID = (TBL // 8) // PBLK

_HI = lax.Precision.HIGHEST


def _pdot(x, w):
  return jnp.dot(x, w, preferred_element_type=jnp.float32, precision=_HI)


def _prelu(h, a):
  return jnp.maximum(h, 0.0) + a * jnp.minimum(h, 0.0)


def _tc1_body(p0, p1, xr, wrA, wrB, wtA, wtB, bA, bB, a, oA, oB):
  agg = p0[...] + p1[...]
  x = xr[...]
  hA = _pdot(agg, wrA[...]) + _pdot(x, wtA[...]) + bA[...]
  hB = _pdot(agg, wrB[...]) + _pdot(x, wtB[...]) + bB[...]
  aa = a[0, 0]
  oA[...] = _prelu(hA, aa)
  oB[...] = _prelu(hB, aa)


_pk_spec = pl.BlockSpec((PBLK, 128), lambda i: (i, 0))
_wd_spec = pl.BlockSpec((128, 128), lambda i: (0, 0))
_bd_spec = pl.BlockSpec((1, 128), lambda i: (0, 0))
_a_spec = pl.BlockSpec((1, 1), lambda i: (0, 0))

_pk_out = [jax.ShapeDtypeStruct((TBL // 8, 128), jnp.float32),
           jax.ShapeDtypeStruct((TBL // 8, 128), jnp.float32)]

_tc_layer1 = pl.pallas_call(
    _tc1_body,
    grid=(PGRID,),
    in_specs=[_pk_spec, _pk_spec, _pk_spec,
              _wd_spec, _wd_spec, _wd_spec, _wd_spec,
              _bd_spec, _bd_spec, _a_spec],
    out_specs=[_pk_spec, _pk_spec],
    out_shape=_pk_out,
)


def _tc2_body(aA, aB, hA, hB, wrAA, wrBA, wtAA, wtBA, wrAB, wrBB, wtAB, wtBB,
              bA, bB, a, oA, oB):
  vA, vB, uA, uB = aA[...], aB[...], hA[...], hB[...]
  h_A = (_pdot(vA, wrAA[...]) + _pdot(vB, wrBA[...])
         + _pdot(uA, wtAA[...]) + _pdot(uB, wtBA[...]) + bA[...])
  h_B = (_pdot(vA, wrAB[...]) + _pdot(vB, wrBB[...])
         + _pdot(uA, wtAB[...]) + _pdot(uB, wtBB[...]) + bB[...])
  aa = a[0, 0]
  oA[...] = _prelu(h_A, aa)
  oB[...] = _prelu(h_B, aa)


_tc_layer2 = pl.pallas_call(
    _tc2_body,
    grid=(PGRID,),
    in_specs=[_pk_spec, _pk_spec, _pk_spec, _pk_spec]
             + [_wd_spec] * 8 + [_bd_spec, _bd_spec, _a_spec],
    out_specs=[_pk_spec, _pk_spec],
    out_shape=_pk_out,
)


def _tc_final_body(u, w3rA, w3rB, w3tA, w3tB, ce, b3t, w0d, b0t, woutd, boutv,
                   out):
  uu = u[...]
  z = jnp.zeros((8, 128), jnp.float32)
  ua, ub, s2a, s2b, cn = z, z, z, z, z
  for t in range(NS):
    ua = ua + uu[t * 9:t * 9 + 8, :]
    ub = ub + uu[432 + t * 9:432 + t * 9 + 8, :]
    s2a = s2a + uu[144 + t * 9:144 + t * 9 + 8, :]
    s2b = s2b + uu[576 + t * 9:576 + t * 9 + 8, :]
    cn = cn + uu[288 + t * 9:288 + t * 9 + 8, :]
  cnt32 = _pdot(cn, ce[...])
  sums3 = (_pdot(ua, w3rA[...]) + _pdot(ub, w3rB[...])
           + _pdot(s2a, w3tA[...]) + _pdot(s2b, w3tB[...])
           + cnt32 * b3t[...])
  pooled = sums3 / jnp.maximum(cnt32, 1.0)
  o = _pdot(pooled, w0d[...]) + b0t[...]
  out[...] = _pdot(o, woutd[...]) + boutv[0, 0]


_w3_spec = pl.BlockSpec((128, 256), lambda i: (0, 0))
_tc_final = pl.pallas_call(
    _tc_final_body,
    grid=(1,),
    in_specs=[pl.BlockSpec((864, 128), lambda i: (0, 0)),
              _w3_spec, _w3_spec, _w3_spec, _w3_spec, _w3_spec,
              pl.BlockSpec((1, 256), lambda i: (0, 0)),
              pl.BlockSpec((256, 256), lambda i: (0, 0)),
              pl.BlockSpec((1, 256), lambda i: (0, 0)),
              pl.BlockSpec((256, 8), lambda i: (0, 0)),
              pl.BlockSpec((1, 1), lambda i: (0, 0))],
    out_specs=pl.BlockSpec((8, 8), lambda i: (0, 0)),
    out_shape=jax.ShapeDtypeStruct((8, 8), jnp.float32),
)


def _bd8(w):
  return jnp.kron(jnp.eye(8, dtype=jnp.float32), w)


def kernel(x, edge_index, batch, W1_rel, W1_root, b1, a1, W2_rel, W2_root, b2,
           a2, W3_rel, W3_root, b3, W0, b0, Wout, bout):
  src = edge_index[0]
  dst = edge_index[1]
  pad = E_PAD - E
  srcp = jnp.concatenate([src, jnp.zeros((pad,), jnp.int32)]).reshape(
      ROWS128, 128)
  dstp = jnp.concatenate([dst, jnp.full((pad,), SINK, jnp.int32)]).reshape(
      ROWS128, 128)
  batchp = jnp.concatenate([batch, jnp.full((BPAD - N,), G, jnp.int32)])

  xp = x.reshape(N // 8, 128)
  a1r = a1.reshape(1, 1)
  a2r = a2.reshape(1, 1)

  w1rA = _bd8(W1_rel[:, :HH])
  w1rB = _bd8(W1_rel[:, HH:])
  w1tA = _bd8(W1_root[:, :HH])
  w1tB = _bd8(W1_root[:, HH:])
  b1A = jnp.tile(b1[:HH], 8).reshape(1, 128)
  b1B = jnp.tile(b1[HH:], 8).reshape(1, 128)
  w2 = [_bd8(Wm[rr, cc])
        for Wm in (W2_rel, W2_root)
        for cc in (slice(0, HH), slice(HH, H))
        for rr in (slice(0, HH), slice(HH, H))]
  b2A = jnp.tile(b2[:HH], 8).reshape(1, 128)
  b2B = jnp.tile(b2[HH:], 8).reshape(1, 128)
  w3rA = _bd8(W3_rel[:HH, :])
  w3rB = _bd8(W3_rel[HH:, :])
  w3tA = _bd8(W3_root[:HH, :])
  w3tB = _bd8(W3_root[HH:, :])
  ce = _bd8(jnp.ones((16, 1), jnp.float32)
            .at[1:, 0].set(0.0) @ jnp.ones((1, 32), jnp.float32))
  b3t = jnp.tile(b3, 8).reshape(1, 256)
  w0d = _bd8(W0)
  b0t = jnp.tile(b0, 8).reshape(1, 256)
  woutd = _bd8(Wout)
  boutv = bout.reshape(1, 1)

  p0, p1 = _segsum_edge(x, x, srcp, dstp)
  h1Ap, h1Bp = _tc_layer1(p0.reshape(TBL // 8, 128), p1.reshape(TBL // 8, 128),
                          xp, w1rA, w1rB, w1tA, w1tB, b1A, b1B, a1r)
  h1A = h1Ap.reshape(TBL, HH)
  h1B = h1Bp.reshape(TBL, HH)
  aggA, aggB = _segsum_col(h1A, h1B, srcp, dstp)
  h2Ap, h2Bp = _tc_layer2(
      aggA.reshape(TBL // 8, 128), aggB.reshape(TBL // 8, 128), h1Ap, h1Bp,
      w2[0], w2[1], w2[4], w2[5], w2[2], w2[3], w2[6], w2[7],
      b2A, b2B, a2r)
  uacc = _pool_sc(h2Ap.reshape(TBL, HH), h2Bp.reshape(TBL, HH),
                  srcp, dstp, batchp)
  o = _tc_final(uacc.reshape((2 * ACC3) // 8, 128),
                w3rA, w3rB, w3tA, w3tB, ce, b3t, w0d, b0t, woutd, boutv)
  return o.reshape(G, 1)


# speedup vs baseline: 25.0288x; 1.3900x over previous
import functools

import jax
import jax.numpy as jnp
from jax import lax
from jax.experimental import pallas as pl
from jax.experimental.pallas import tpu as pltpu
from jax.experimental.pallas import tpu_sc as plsc

N = 100000
E = 1600000
F_IN = 16
H = 32
HH = 16
G = 64

NC = 2
NS = 16

E_PAD = 32 * 49 * 1024
ROWS128 = E_PAD // 128
EPW = E_PAD // 32
EPC = E_PAD // 16
SINK = N

TBL = 102400
ZCH = 320

TPT = 72
REG = NS * TPT
ACC3 = 3 * REG
BPAD = EPC
NPT = BPAD // 16

_mesh = plsc.VectorSubcoreMesh(
    core_axis_name="c", subcore_axis_name="s", num_cores=NC, num_subcores=NS)


def _fill(ref, nrows, width, val):
  v = jnp.full((width,), val, jnp.float32)
  def body(i, carry):
    ref[i] = v
    return carry
  lax.fori_loop(0, nrows, body, 0)


def _make_segsum(edge_split):
  n_sup = (EPW if edge_split else EPC) // 512

  @functools.partial(
      pl.kernel,
      mesh=_mesh,
      compiler_params=pltpu.CompilerParams(use_tc_tiling_on_sc=False),
      out_type=[jax.ShapeDtypeStruct((TBL, HH), jnp.float32),
                jax.ShapeDtypeStruct((TBL, HH), jnp.float32)],
      scratch_types=[
          pltpu.VMEM((2, 4, 128), jnp.int32),
          pltpu.VMEM((2, 4, 128), jnp.int32),
          pltpu.VMEM((2, 4, 128, HH), jnp.float32),
          pltpu.VMEM((ZCH, HH), jnp.float32),
          pltpu.SemaphoreType.DMA,
          pltpu.SemaphoreType.DMA,
          pltpu.SemaphoreType.DMA,
          pltpu.VMEM_SHARED((TBL, HH), jnp.float32),
      ],
  )
  def k(tA, tB, src2d, dst2d, outA, outB, svec, dvec, rows, zbuf,
        gsem, ssem, isem, table):
    c = lax.axis_index("c")
    s = lax.axis_index("s")

    _fill(zbuf, ZCH, HH, 0.0)
    for kk in range(20):
      pltpu.sync_copy(zbuf, table.at[pl.ds(s * 6400 + kk * ZCH, ZCH)])
    plsc.subcore_barrier()

    if edge_split:
      base128 = (s * NC + c) * (EPW // 128)
    else:
      base128 = s * (EPC // 128)

    def fire_idx(p, i):
      rb = base128 + i * 4
      pltpu.async_copy(src2d.at[pl.ds(rb, 4)], svec.at[p], isem)
      pltpu.async_copy(dst2d.at[pl.ds(rb, 4)], dvec.at[p], isem)

    def wait_idx(p, i):
      rb = base128 + i * 4
      pltpu.make_async_copy(src2d.at[pl.ds(rb, 4)], svec.at[p], isem).wait()
      pltpu.make_async_copy(dst2d.at[pl.ds(rb, 4)], dvec.at[p], isem).wait()

    def fire_gathers(p):
      for j in range(4):
        @pl.when(c == 0)
        def _g0():
          pltpu.async_copy(tA.at[svec.at[p].at[j]], rows.at[p].at[j], gsem)
        @pl.when(c == 1)
        def _g1():
          pltpu.async_copy(tB.at[svec.at[p].at[j]], rows.at[p].at[j], gsem)

    def wait_gathers(p):
      for j in range(4):
        pltpu.make_async_copy(
            tA.at[svec.at[p].at[j]], rows.at[p].at[j], gsem).wait()

    def fire_scatters(p):
      for j in range(4):
        pltpu.async_copy(rows.at[p].at[j], table.at[dvec.at[p].at[j]], ssem,
                         add=True)

    def wait_scatters(p):
      for j in range(4):
        pltpu.make_async_copy(
            rows.at[p].at[j], table.at[dvec.at[p].at[j]], ssem).wait()

    def process(p, i):
      q = 1 - p
      @pl.when(i >= 1)
      def _dr():
        wait_scatters(q)
      @pl.when(i + 1 < n_sup)
      def _pf():
        fire_idx(q, i + 1)
      wait_gathers(p)
      fire_scatters(p)
      @pl.when(i + 1 < n_sup)
      def _nx():
        wait_idx(q, i + 1)
        fire_gathers(q)

    fire_idx(0, 0)
    wait_idx(0, 0)
    fire_gathers(0)

    def body(i, carry):
      @pl.when(i % 2 == 0)
      def _p0():
        process(0, i)
      @pl.when(i % 2 == 1)
      def _p1():
        process(1, i)
      return carry

    lax.fori_loop(0, n_sup, body, 0)
    wait_scatters((n_sup - 1) % 2)
    plsc.subcore_barrier()

    for kk in range(20):
      off = s * 6400 + kk * ZCH
      pltpu.sync_copy(table.at[pl.ds(off, ZCH)], zbuf)
      @pl.when(c == 0)
      def _w0():
        pltpu.sync_copy(zbuf, outA.at[pl.ds(off, ZCH)])
      @pl.when(c == 1)
      def _w1():
        pltpu.sync_copy(zbuf, outB.at[pl.ds(off, ZCH)])

  return k


_segsum_edge = _make_segsum(True)
_segsum_col = _make_segsum(False)


@functools.partial(
    pl.kernel,
    mesh=_mesh,
    compiler_params=pltpu.CompilerParams(use_tc_tiling_on_sc=False),
    out_type=jax.ShapeDtypeStruct((2 * ACC3, HH), jnp.float32),
    scratch_types=[
        pltpu.VMEM((2, 8, 128), jnp.int32),
        pltpu.VMEM((2, 8, 128), jnp.int32),
        pltpu.VMEM((2, 8, 128, HH), jnp.float32),
        pltpu.VMEM((2, 8, 128), jnp.int32),
        pltpu.VMEM((NPT,), jnp.int32),
        pltpu.VMEM((TPT, HH), jnp.float32),
        pltpu.VMEM((128, HH), jnp.float32),
        pltpu.VMEM((128, HH), jnp.float32),
        pltpu.VMEM((128,), jnp.int32),
        pltpu.SemaphoreType.DMA,
        pltpu.SemaphoreType.DMA,
        pltpu.SemaphoreType.DMA,
        pltpu.SemaphoreType.DMA,
        pltpu.VMEM_SHARED((BPAD,), jnp.int32),
        pltpu.VMEM_SHARED((ACC3, HH), jnp.float32),
    ],
)
def _pool_sc(tA, tB, src2d, dst2d, batch_hbm, out, svec, dvec, rows, bdrow,
             bbuf, zobuf, nrow, ones, nidx, gsem, bsem, ssem, isem,
             batch_sp, acc):
  c = lax.axis_index("c")
  s = lax.axis_index("s")
  s72 = s * TPT

  _fill(zobuf, TPT, HH, 0.0)
  for r in range(3):
    pltpu.sync_copy(zobuf, acc.at[pl.ds(r * REG + s72, TPT)])
  _fill(ones, 128, HH, 1.0)
  pltpu.sync_copy(batch_hbm.at[pl.ds(s * NPT, NPT)], bbuf)
  pltpu.sync_copy(bbuf, batch_sp.at[pl.ds(s * NPT, NPT)])
  plsc.subcore_barrier()

  base128 = s * (EPC // 128)
  n_sup = EPC // 1024

  def fire_idx(p, i):
    rb = base128 + i * 8
    pltpu.async_copy(src2d.at[pl.ds(rb, 8)], svec.at[p], isem)
    pltpu.async_copy(dst2d.at[pl.ds(rb, 8)], dvec.at[p], isem)

  def wait_idx(p, i):
    rb = base128 + i * 8
    pltpu.make_async_copy(src2d.at[pl.ds(rb, 8)], svec.at[p], isem).wait()
    pltpu.make_async_copy(dst2d.at[pl.ds(rb, 8)], dvec.at[p], isem).wait()

  def fire_gathers(p):
    for j in range(8):
      pltpu.async_copy(batch_sp.at[dvec.at[p].at[j]], bdrow.at[p].at[j], bsem)
    for j in range(8):
      @pl.when(c == 0)
      def _g0():
        pltpu.async_copy(tA.at[svec.at[p].at[j]], rows.at[p].at[j], gsem)
      @pl.when(c == 1)
      def _g1():
        pltpu.async_copy(tB.at[svec.at[p].at[j]], rows.at[p].at[j], gsem)

  def wait_scatters(p):
    for j in range(8):
      pltpu.make_async_copy(
          rows.at[p].at[j], acc.at[bdrow.at[p].at[j]], ssem).wait()

  def process(p, i):
    q = 1 - p
    @pl.when(i >= 1)
    def _dr():
      wait_scatters(q)
    @pl.when(i + 1 < n_sup)
    def _pf():
      fire_idx(q, i + 1)
    for j in range(8):
      pltpu.make_async_copy(
          batch_sp.at[dvec.at[p].at[j]], bdrow.at[p].at[j], bsem).wait()
    for j in range(8):
      for qq in range(8):
        bdrow[p, j, pl.ds(qq * 16, 16)] = (
            bdrow[p, j, pl.ds(qq * 16, 16)] + s72)
    for j in range(8):
      pltpu.make_async_copy(
          tA.at[svec.at[p].at[j]], rows.at[p].at[j], gsem).wait()
    for j in range(8):
      pltpu.async_copy(rows.at[p].at[j], acc.at[bdrow.at[p].at[j]], ssem,
                       add=True)
    @pl.when(i + 1 < n_sup)
    def _nx():
      wait_idx(q, i + 1)
      fire_gathers(q)

  fire_idx(0, 0)
  wait_idx(0, 0)
  fire_gathers(0)

  def body(i, carry):
    @pl.when(i % 2 == 0)
    def _p0():
      process(0, i)
    @pl.when(i % 2 == 1)
    def _p1():
      process(1, i)
    return carry

  lax.fori_loop(0, n_sup, body, 0)
  wait_scatters((n_sup - 1) % 2)

  def npass(k, carry):
    nb = s * NPT + k * 128
    @pl.when(c == 0)
    def _n0():
      pltpu.sync_copy(tA.at[pl.ds(nb, 128)], nrow)
    @pl.when(c == 1)
    def _n1():
      pltpu.sync_copy(tB.at[pl.ds(nb, 128)], nrow)
    for q in range(8):
      nidx[pl.ds(q * 16, 16)] = (
          bbuf[pl.ds(k * 128 + q * 16, 16)] + (REG + s72))
    pltpu.sync_copy(nrow, acc.at[nidx], add=True)
    for q in range(8):
      nidx[pl.ds(q * 16, 16)] = nidx[pl.ds(q * 16, 16)] + REG
    pltpu.sync_copy(ones, acc.at[nidx], add=True)
    return carry

  lax.fori_loop(0, NPT // 128, npass, 0)
  plsc.subcore_barrier()

  for r in range(3):
    pltpu.sync_copy(acc.at[pl.ds(r * REG + s72, TPT)], zobuf)
    pltpu.sync_copy(zobuf, out.at[pl.ds(c * ACC3 + r * REG + s72, TPT)])


PBLK = 1280
PGRID = (TBL // 8) // PBLK

_HI = lax.Precision.HIGHEST


def _pdot(x, w):
  return jnp.dot(x, w, preferred_element_type=jnp.float32, precision=_HI)


def _prelu(h, a):
  return jnp.maximum(h, 0.0) + a * jnp.minimum(h, 0.0)


def _tc1_body(p0, p1, xr, wrA, wrB, wtA, wtB, bA, bB, a, oA, oB):
  agg = p0[...] + p1[...]
  x = xr[...]
  hA = _pdot(agg, wrA[...]) + _pdot(x, wtA[...]) + bA[...]
  hB = _pdot(agg, wrB[...]) + _pdot(x, wtB[...]) + bB[...]
  aa = a[0, 0]
  oA[...] = _prelu(hA, aa)
  oB[...] = _prelu(hB, aa)


_pk_spec = pl.BlockSpec((PBLK, 128), lambda i: (i, 0))
_wd_spec = pl.BlockSpec((128, 128), lambda i: (0, 0))
_bd_spec = pl.BlockSpec((1, 128), lambda i: (0, 0))
_a_spec = pl.BlockSpec((1, 1), lambda i: (0, 0))

_pk_out = [jax.ShapeDtypeStruct((TBL // 8, 128), jnp.float32),
           jax.ShapeDtypeStruct((TBL // 8, 128), jnp.float32)]

_tc_layer1 = pl.pallas_call(
    _tc1_body,
    grid=(PGRID,),
    in_specs=[_pk_spec, _pk_spec, _pk_spec,
              _wd_spec, _wd_spec, _wd_spec, _wd_spec,
              _bd_spec, _bd_spec, _a_spec],
    out_specs=[_pk_spec, _pk_spec],
    out_shape=_pk_out,
)


def _tc2_body(aA, aB, hA, hB, wrAA, wrBA, wtAA, wtBA, wrAB, wrBB, wtAB, wtBB,
              bA, bB, a, oA, oB):
  vA, vB, uA, uB = aA[...], aB[...], hA[...], hB[...]
  h_A = (_pdot(vA, wrAA[...]) + _pdot(vB, wrBA[...])
         + _pdot(uA, wtAA[...]) + _pdot(uB, wtBA[...]) + bA[...])
  h_B = (_pdot(vA, wrAB[...]) + _pdot(vB, wrBB[...])
         + _pdot(uA, wtAB[...]) + _pdot(uB, wtBB[...]) + bB[...])
  aa = a[0, 0]
  oA[...] = _prelu(h_A, aa)
  oB[...] = _prelu(h_B, aa)


_tc_layer2 = pl.pallas_call(
    _tc2_body,
    grid=(PGRID,),
    in_specs=[_pk_spec, _pk_spec, _pk_spec, _pk_spec]
             + [_wd_spec] * 8 + [_bd_spec, _bd_spec, _a_spec],
    out_specs=[_pk_spec, _pk_spec],
    out_shape=_pk_out,
)


def _tc_final_body(u, w3rA, w3rB, w3tA, w3tB, ce, b3t, w0d, b0t, woutd, boutv,
                   out):
  uu = u[...]
  z = jnp.zeros((8, 128), jnp.float32)
  ua, ub, s2a, s2b, cn = z, z, z, z, z
  for t in range(NS):
    ua = ua + uu[t * 9:t * 9 + 8, :]
    ub = ub + uu[432 + t * 9:432 + t * 9 + 8, :]
    s2a = s2a + uu[144 + t * 9:144 + t * 9 + 8, :]
    s2b = s2b + uu[576 + t * 9:576 + t * 9 + 8, :]
    cn = cn + uu[288 + t * 9:288 + t * 9 + 8, :]
  cnt32 = _pdot(cn, ce[...])
  sums3 = (_pdot(ua, w3rA[...]) + _pdot(ub, w3rB[...])
           + _pdot(s2a, w3tA[...]) + _pdot(s2b, w3tB[...])
           + cnt32 * b3t[...])
  pooled = sums3 / jnp.maximum(cnt32, 1.0)
  o = _pdot(pooled, w0d[...]) + b0t[...]
  out[...] = _pdot(o, woutd[...]) + boutv[0, 0]


_w3_spec = pl.BlockSpec((128, 256), lambda i: (0, 0))
_tc_final = pl.pallas_call(
    _tc_final_body,
    grid=(1,),
    in_specs=[pl.BlockSpec((864, 128), lambda i: (0, 0)),
              _w3_spec, _w3_spec, _w3_spec, _w3_spec, _w3_spec,
              pl.BlockSpec((1, 256), lambda i: (0, 0)),
              pl.BlockSpec((256, 256), lambda i: (0, 0)),
              pl.BlockSpec((1, 256), lambda i: (0, 0)),
              pl.BlockSpec((256, 8), lambda i: (0, 0)),
              pl.BlockSpec((1, 1), lambda i: (0, 0))],
    out_specs=pl.BlockSpec((8, 8), lambda i: (0, 0)),
    out_shape=jax.ShapeDtypeStruct((8, 8), jnp.float32),
)


def _bd8(w):
  return jnp.kron(jnp.eye(8, dtype=jnp.float32), w)


def kernel(x, edge_index, batch, W1_rel, W1_root, b1, a1, W2_rel, W2_root, b2,
           a2, W3_rel, W3_root, b3, W0, b0, Wout, bout):
  src = edge_index[0]
  dst = edge_index[1]
  pad = E_PAD - E
  srcp = jnp.concatenate([src, jnp.zeros((pad,), jnp.int32)]).reshape(
      ROWS128, 128)
  dstp = jnp.concatenate([dst, jnp.full((pad,), SINK, jnp.int32)]).reshape(
      ROWS128, 128)
  batchp = jnp.concatenate([batch, jnp.full((BPAD - N,), G, jnp.int32)])

  xp = x.reshape(N // 8, 128)
  a1r = a1.reshape(1, 1)
  a2r = a2.reshape(1, 1)

  w1rA = _bd8(W1_rel[:, :HH])
  w1rB = _bd8(W1_rel[:, HH:])
  w1tA = _bd8(W1_root[:, :HH])
  w1tB = _bd8(W1_root[:, HH:])
  b1A = jnp.tile(b1[:HH], 8).reshape(1, 128)
  b1B = jnp.tile(b1[HH:], 8).reshape(1, 128)
  w2 = [_bd8(Wm[rr, cc])
        for Wm in (W2_rel, W2_root)
        for cc in (slice(0, HH), slice(HH, H))
        for rr in (slice(0, HH), slice(HH, H))]
  b2A = jnp.tile(b2[:HH], 8).reshape(1, 128)
  b2B = jnp.tile(b2[HH:], 8).reshape(1, 128)
  w3rA = _bd8(W3_rel[:HH, :])
  w3rB = _bd8(W3_rel[HH:, :])
  w3tA = _bd8(W3_root[:HH, :])
  w3tB = _bd8(W3_root[HH:, :])
  ce = _bd8(jnp.ones((16, 1), jnp.float32)
            .at[1:, 0].set(0.0) @ jnp.ones((1, 32), jnp.float32))
  b3t = jnp.tile(b3, 8).reshape(1, 256)
  w0d = _bd8(W0)
  b0t = jnp.tile(b0, 8).reshape(1, 256)
  woutd = _bd8(Wout)
  boutv = bout.reshape(1, 1)

  p0, p1 = _segsum_edge(x, x, srcp, dstp)
  h1Ap, h1Bp = _tc_layer1(p0.reshape(TBL // 8, 128), p1.reshape(TBL // 8, 128),
                          xp, w1rA, w1rB, w1tA, w1tB, b1A, b1B, a1r)
  h1A = h1Ap.reshape(TBL, HH)
  h1B = h1Bp.reshape(TBL, HH)
  aggA, aggB = _segsum_col(h1A, h1B, srcp, dstp)
  h2Ap, h2Bp = _tc_layer2(
      aggA.reshape(TBL // 8, 128), aggB.reshape(TBL // 8, 128), h1Ap, h1Bp,
      w2[0], w2[1], w2[4], w2[5], w2[2], w2[3], w2[6], w2[7],
      b2A, b2B, a2r)
  uacc = _pool_sc(h2Ap.reshape(TBL, HH), h2Bp.reshape(TBL, HH),
                  srcp, dstp, batchp)
  o = _tc_final(uacc.reshape((2 * ACC3) // 8, 128),
                w3rA, w3rB, w3tA, w3tB, ce, b3t, w0d, b0t, woutd, boutv)
  return o.reshape(G, 1)
